# baseline (speedup 1.0000x reference)
import functools

import jax
import jax.numpy as jnp
from jax import lax
from jax.experimental import pallas as pl
from jax.experimental.pallas import tpu as pltpu
from jax.experimental.pallas import tpu_sc as plsc

B, C_IN, C_OUT, N, K = 4, 256, 256, 4096, 16
ROWS = B * N
BH = B // 2
HROWS = BH * N
NW = 32
RPW = HROWS // NW
G = 8
NBLK = RPW // G
NBUF = 4
LANES = 16
NB = 1024


def _sc_body(xt, idxg, xj, idx_all, *bufs):
    rows_v = bufs[0:NBUF]
    acc_v = bufs[NBUF:2 * NBUF]
    sem_g = bufs[2 * NBUF:3 * NBUF]
    sem_o = bufs[3 * NBUF:4 * NBUF]
    wid = lax.axis_index("s") * 2 + lax.axis_index("c")
    base = wid * RPW

    pltpu.sync_copy(idxg.at[pl.ds(base * K, RPW * K)], idx_all)

    boff = lax.broadcast((wid // (NW // BH)) * N, (LANES,))

    def rebase(i, carry):
        sl = pl.ds(i * LANES, LANES)
        idx_all[sl] = idx_all[sl] + boff
        return carry

    lax.fori_loop(0, RPW * K // LANES, rebase, 0)

    def start_gather(i, b):
        pltpu.async_copy(
            xt.at[idx_all.at[pl.ds(i * G * K, G * K)]], rows_v[b], sem_g[b])

    def out_slice(i):
        return xj.at[pl.ds(base + i * G, G)]

    for b in range(NBUF):
        start_gather(b, b)

    def do_block(i, b):
        @pl.when(i >= NBUF)
        def _():
            pltpu.make_async_copy(acc_v[b], out_slice(i - NBUF),
                                  sem_o[b]).wait()

        pltpu.make_async_copy(xt.at[idx_all.at[pl.ds(i * G * K, G * K)]],
                              rows_v[b], sem_g[b]).wait()

        def node(g, carry):
            for m in range(C_IN // (2 * LANES)):
                sl = pl.ds(m * LANES, LANES)

                def row(k):
                    w = rows_v[b][g * K + k, sl]
                    return plsc.unpack(plsc.bitcast(w, jnp.bfloat16),
                                       format=plsc.PackFormat.INTERLEAVED)

                pa, pb = row(0)
                for k in range(1, K):
                    qa, qb = row(k)
                    pa = pa + qa
                    pb = pb + qb
                packed = plsc.pack(pa, pb, format=plsc.PackFormat.INTERLEAVED)
                acc_v[b][g, sl] = plsc.bitcast(packed, jnp.int32)
            return carry

        lax.fori_loop(0, G, node, 0)
        pltpu.async_copy(acc_v[b], out_slice(i), sem_o[b])

        @pl.when(i + NBUF < NBLK)
        def _():
            start_gather(i + NBUF, b)

    def blk(j, carry):
        for b in range(NBUF):
            do_block(j * NBUF + b, b)
        return carry

    lax.fori_loop(0, NBLK // NBUF, blk, 0)
    for b in range(NBUF):
        pltpu.make_async_copy(acc_v[b], out_slice(NBLK - NBUF + b),
                              sem_o[b]).wait()


_sc_gather = functools.partial(
    pl.kernel,
    mesh=plsc.VectorSubcoreMesh(core_axis_name="c", subcore_axis_name="s"),
    compiler_params=pltpu.CompilerParams(needs_layout_passes=False),
    out_type=jax.ShapeDtypeStruct((HROWS, C_IN // 2), jnp.int32),
    scratch_types=(
        [pltpu.VMEM((RPW * K,), jnp.int32)]
        + [pltpu.VMEM((G * K, C_IN // 2), jnp.int32)] * NBUF
        + [pltpu.VMEM((G, C_IN // 2), jnp.int32)] * NBUF
        + [pltpu.SemaphoreType.DMA] * (2 * NBUF)
    ),
)(_sc_body)


NBT = 2048


def _tr_body(x_ref, o_ref):
    xb = x_ref[0].T.astype(jnp.bfloat16)
    lo = lax.bitcast_convert_type(xb[:, :C_IN // 2], jnp.uint16)
    hi = lax.bitcast_convert_type(xb[:, C_IN // 2:], jnp.uint16)
    word = lo.astype(jnp.uint32) | (hi.astype(jnp.uint32) << 16)
    o_ref[...] = lax.bitcast_convert_type(word, jnp.int32)


def _tc_transpose(x3):
    return pl.pallas_call(
        _tr_body,
        grid=(N // NBT, BH),
        in_specs=[pl.BlockSpec((1, C_IN, NBT), lambda i, b: (b, 0, i))],
        out_specs=pl.BlockSpec((NBT, C_IN // 2),
                               lambda i, b: (b * (N // NBT) + i, 0)),
        out_shape=jax.ShapeDtypeStruct((HROWS, C_IN // 2), jnp.int32),
    )(x3)


def _tc_body(x_ref, xj_ref, we_ref, w_ref, b_ref, o_ref):
    xb = x_ref[0]
    xju = lax.bitcast_convert_type(xj_ref[...], jnp.uint32)
    lo = lax.bitcast_convert_type(xju << 16, jnp.float32)
    hi = lax.bitcast_convert_type(xju & jnp.uint32(0xFFFF0000),
                                  jnp.float32)
    s1 = lax.dot_general(we_ref[...], xb, (((1,), (0,)), ((), ())),
                         preferred_element_type=jnp.float32)
    w = w_ref[...]
    s2 = lax.dot_general(w[:, :C_IN // 2], lo, (((1,), (1,)), ((), ())),
                         preferred_element_type=jnp.float32)
    s3 = lax.dot_general(w[:, C_IN // 2:], hi, (((1,), (1,)), ((), ())),
                         preferred_element_type=jnp.float32)
    o_ref[...] = jnp.maximum(s1 + s2 + s3 + b_ref[...], 0.0)[None]


def _tc_conv(x3, xj, W_eps, W, b2):
    nblk = N // NB
    return pl.pallas_call(
        _tc_body,
        grid=(HROWS // NB,),
        in_specs=[
            pl.BlockSpec((1, C_IN, NB), lambda i: (i // nblk, 0, i % nblk)),
            pl.BlockSpec((NB, C_IN // 2), lambda i: (i, 0)),
            pl.BlockSpec((C_OUT, C_IN), lambda i: (0, 0)),
            pl.BlockSpec((C_OUT, C_IN), lambda i: (0, 0)),
            pl.BlockSpec((C_OUT, 1), lambda i: (0, 0)),
        ],
        out_specs=pl.BlockSpec((1, C_OUT, NB),
                               lambda i: (i // nblk, 0, i % nblk)),
        out_shape=jax.ShapeDtypeStruct((BH, C_OUT, N), jnp.float32),
    )(x3, xj, W_eps, W, b2)


def kernel(x, edge_index, W, bconv, eps):
    x3 = x.reshape(B, C_IN, N)
    W_eps = (1.0 + eps[0]) * W
    b2 = bconv.reshape(C_OUT, 1)
    outs = []
    for h in range(B // BH):
        xh = x3[h * BH:(h + 1) * BH]
        idxh = edge_index[0, h * BH:(h + 1) * BH].reshape(-1)
        xth = _tc_transpose(xh)
        xjh = _sc_gather(xth, idxh)
        outs.append(_tc_conv(xh, xjh, W_eps, W, b2))
    out = jnp.concatenate(outs, axis=0)
    return out.reshape(B, C_OUT, N, 1)


# speedup vs baseline: 1.1305x; 1.1305x over previous
import functools

import jax
import jax.numpy as jnp
from jax import lax
from jax.experimental import pallas as pl
from jax.experimental.pallas import tpu as pltpu
from jax.experimental.pallas import tpu_sc as plsc

B, C_IN, C_OUT, N, K = 4, 256, 256, 4096, 16
ROWS = B * N
NW = 32
RPW = ROWS // NW
G = 8
NBLK = RPW // G
NBUF = 4
LANES = 16
NB = 1024


def _sc_body(xt, idxg, xj, idx_all, *bufs):
    rows_v = bufs[0:NBUF]
    acc_v = bufs[NBUF:2 * NBUF]
    sem_g = bufs[2 * NBUF:3 * NBUF]
    sem_o = bufs[3 * NBUF:4 * NBUF]
    wid = lax.axis_index("s") * 2 + lax.axis_index("c")
    base = wid * RPW

    pltpu.sync_copy(idxg.at[pl.ds(base * K, RPW * K)], idx_all)

    boff = lax.broadcast((wid // (NW // B)) * N, (LANES,))

    def rebase(i, carry):
        sl = pl.ds(i * LANES, LANES)
        idx_all[sl] = idx_all[sl] + boff
        return carry

    lax.fori_loop(0, RPW * K // LANES, rebase, 0)

    def start_gather(i, b):
        pltpu.async_copy(
            xt.at[idx_all.at[pl.ds(i * G * K, G * K)]], rows_v[b], sem_g[b])

    def out_slice(i):
        return xj.at[pl.ds(base + i * G, G)]

    for b in range(NBUF):
        start_gather(b, b)

    def do_block(i, b):
        @pl.when(i >= NBUF)
        def _():
            pltpu.make_async_copy(acc_v[b], out_slice(i - NBUF),
                                  sem_o[b]).wait()

        pltpu.make_async_copy(xt.at[idx_all.at[pl.ds(i * G * K, G * K)]],
                              rows_v[b], sem_g[b]).wait()

        def node(g, carry):
            for m in range(C_IN // (2 * LANES)):
                sl = pl.ds(m * LANES, LANES)

                def row(k):
                    w = rows_v[b][g * K + k, sl]
                    return plsc.unpack(plsc.bitcast(w, jnp.bfloat16),
                                       format=plsc.PackFormat.INTERLEAVED)

                pa, pb = row(0)
                for k in range(1, K):
                    qa, qb = row(k)
                    pa = pa + qa
                    pb = pb + qb
                packed = plsc.pack(pa, pb, format=plsc.PackFormat.INTERLEAVED)
                acc_v[b][g, sl] = plsc.bitcast(packed, jnp.int32)
            return carry

        lax.fori_loop(0, G, node, 0)
        pltpu.async_copy(acc_v[b], out_slice(i), sem_o[b])

        @pl.when(i + NBUF < NBLK)
        def _():
            start_gather(i + NBUF, b)

    def blk(j, carry):
        for b in range(NBUF):
            do_block(j * NBUF + b, b)
        return carry

    lax.fori_loop(0, NBLK // NBUF, blk, 0)
    for b in range(NBUF):
        pltpu.make_async_copy(acc_v[b], out_slice(NBLK - NBUF + b),
                              sem_o[b]).wait()


_sc_gather = functools.partial(
    pl.kernel,
    mesh=plsc.VectorSubcoreMesh(core_axis_name="c", subcore_axis_name="s"),
    compiler_params=pltpu.CompilerParams(needs_layout_passes=False),
    out_type=jax.ShapeDtypeStruct((ROWS, C_IN // 2), jnp.int32),
    scratch_types=(
        [pltpu.VMEM((RPW * K,), jnp.int32)]
        + [pltpu.VMEM((G * K, C_IN // 2), jnp.int32)] * NBUF
        + [pltpu.VMEM((G, C_IN // 2), jnp.int32)] * NBUF
        + [pltpu.SemaphoreType.DMA] * (2 * NBUF)
    ),
)(_sc_body)


NBT = 2048


def _tr_body(x_ref, o_ref):
    xb = x_ref[0].T.astype(jnp.bfloat16)
    lo = lax.bitcast_convert_type(xb[:, :C_IN // 2], jnp.uint16)
    hi = lax.bitcast_convert_type(xb[:, C_IN // 2:], jnp.uint16)
    word = lo.astype(jnp.uint32) | (hi.astype(jnp.uint32) << 16)
    o_ref[...] = lax.bitcast_convert_type(word, jnp.int32)


def _tc_transpose(x3):
    return pl.pallas_call(
        _tr_body,
        grid=(N // NBT, B),
        in_specs=[pl.BlockSpec((1, C_IN, NBT), lambda i, b: (b, 0, i))],
        out_specs=pl.BlockSpec((NBT, C_IN // 2),
                               lambda i, b: (b * (N // NBT) + i, 0)),
        out_shape=jax.ShapeDtypeStruct((ROWS, C_IN // 2), jnp.int32),
    )(x3)


def _split_bf16(ref):
    u = lax.bitcast_convert_type(ref[...], jnp.uint32)
    lo = lax.bitcast_convert_type(u << 16, jnp.float32)
    hi = lax.bitcast_convert_type(u & jnp.uint32(0xFFFF0000),
                                  jnp.float32)
    return lo, hi


def _half_dot(w_half, v):
    return lax.dot_general(w_half, v, (((1,), (1,)), ((), ())),
                           preferred_element_type=jnp.float32)


def _tc_body(xt_ref, xj_ref, we_ref, w_ref, b_ref, o_ref):
    xlo, xhi = _split_bf16(xt_ref)
    jlo, jhi = _split_bf16(xj_ref)
    we = we_ref[...]
    w = w_ref[...]
    s = (_half_dot(we[:, :C_IN // 2], xlo) + _half_dot(we[:, C_IN // 2:], xhi)
         + _half_dot(w[:, :C_IN // 2], jlo) + _half_dot(w[:, C_IN // 2:], jhi))
    o_ref[...] = jnp.maximum(s + b_ref[...], 0.0)[None]


def _tc_conv(xt, xj, W_eps, W, b2):
    nblk = N // NB
    return pl.pallas_call(
        _tc_body,
        grid=(ROWS // NB,),
        in_specs=[
            pl.BlockSpec((NB, C_IN // 2), lambda i: (i, 0)),
            pl.BlockSpec((NB, C_IN // 2), lambda i: (i, 0)),
            pl.BlockSpec((C_OUT, C_IN), lambda i: (0, 0)),
            pl.BlockSpec((C_OUT, C_IN), lambda i: (0, 0)),
            pl.BlockSpec((C_OUT, 1), lambda i: (0, 0)),
        ],
        out_specs=pl.BlockSpec((1, C_OUT, NB),
                               lambda i: (i // nblk, 0, i % nblk)),
        out_shape=jax.ShapeDtypeStruct((B, C_OUT, N), jnp.float32),
    )(xt, xj, W_eps, W, b2)


def kernel(x, edge_index, W, bconv, eps):
    xt = _tc_transpose(x.reshape(B, C_IN, N))
    idxg = edge_index[0].reshape(-1)
    xj = _sc_gather(xt, idxg)
    W_eps = (1.0 + eps[0]) * W
    out = _tc_conv(xt, xj, W_eps, W, bconv.reshape(C_OUT, 1))
    return out.reshape(B, C_OUT, N, 1)
